# baseline (speedup 1.0000x reference)
import functools

import jax
import jax.numpy as jnp
from jax.experimental import pallas as pl
from jax.experimental.pallas import tpu as pltpu


_CHUNK_ROWS = 512
_NBUF = 3


def _pipeline_kernel(emb_ref, out_ref, *rest, n_chunks, scale):
    ibufs = rest[:_NBUF]
    obufs = rest[_NBUF : 2 * _NBUF]
    isems = rest[2 * _NBUF : 3 * _NBUF]
    osems = rest[3 * _NBUF :]

    in_descs = [None] * n_chunks
    out_descs = [None] * n_chunks

    def fire_in(ci):
        b = ci % _NBUF
        in_descs[ci] = pltpu.make_async_copy(
            emb_ref.at[pl.ds(ci * _CHUNK_ROWS, _CHUNK_ROWS), :], ibufs[b], isems[b]
        )
        in_descs[ci].start()

    for ci in range(min(_NBUF, n_chunks)):
        fire_in(ci)

    for ci in range(n_chunks):
        b = ci % _NBUF
        in_descs[ci].wait()
        if ci >= _NBUF:
            out_descs[ci - _NBUF].wait()
        obufs[b][...] = ibufs[b][...] * scale
        out_descs[ci] = pltpu.make_async_copy(
            obufs[b], out_ref.at[pl.ds(ci * _CHUNK_ROWS, _CHUNK_ROWS), :], osems[b]
        )
        out_descs[ci].start()
        if ci + _NBUF < n_chunks:
            fire_in(ci + _NBUF)

    for ci in range(max(0, n_chunks - _NBUF), n_chunks):
        out_descs[ci].wait()


def kernel(x, emb):
    seq_len = x.shape[1]
    dim = emb.shape[1]
    scale = float(dim) ** -0.5
    table = emb[:seq_len]
    rows = table.shape[0]
    n_chunks = rows // _CHUNK_ROWS
    body = functools.partial(_pipeline_kernel, n_chunks=n_chunks, scale=scale)
    return pl.pallas_call(
        body,
        in_specs=[pl.BlockSpec(memory_space=pl.ANY)],
        out_specs=pl.BlockSpec(memory_space=pl.ANY),
        out_shape=jax.ShapeDtypeStruct((rows, dim), emb.dtype),
        scratch_shapes=(
            [pltpu.VMEM((_CHUNK_ROWS, dim), jnp.float32)] * (2 * _NBUF)
            + [pltpu.SemaphoreType.DMA] * (2 * _NBUF)
        ),
    )(table)


# speedup vs baseline: 1.0506x; 1.0506x over previous
import functools

import jax
import jax.numpy as jnp
from jax.experimental import pallas as pl
from jax.experimental.pallas import tpu as pltpu


_CHUNK_ROWS = 2048
_NBUF = 2


def _pipeline_kernel(emb_ref, out_ref, *rest, n_chunks, scale):
    ibufs = rest[:_NBUF]
    obufs = rest[_NBUF : 2 * _NBUF]
    isems = rest[2 * _NBUF : 3 * _NBUF]
    osems = rest[3 * _NBUF :]

    in_descs = [None] * n_chunks
    out_descs = [None] * n_chunks

    def fire_in(ci):
        b = ci % _NBUF
        in_descs[ci] = pltpu.make_async_copy(
            emb_ref.at[pl.ds(ci * _CHUNK_ROWS, _CHUNK_ROWS), :], ibufs[b], isems[b]
        )
        in_descs[ci].start()

    for ci in range(min(_NBUF, n_chunks)):
        fire_in(ci)

    for ci in range(n_chunks):
        b = ci % _NBUF
        in_descs[ci].wait()
        if ci >= _NBUF:
            out_descs[ci - _NBUF].wait()
        obufs[b][...] = ibufs[b][...] * scale
        out_descs[ci] = pltpu.make_async_copy(
            obufs[b], out_ref.at[pl.ds(ci * _CHUNK_ROWS, _CHUNK_ROWS), :], osems[b]
        )
        out_descs[ci].start()
        if ci + _NBUF < n_chunks:
            fire_in(ci + _NBUF)

    for ci in range(max(0, n_chunks - _NBUF), n_chunks):
        out_descs[ci].wait()


def kernel(x, emb):
    seq_len = x.shape[1]
    dim = emb.shape[1]
    scale = float(dim) ** -0.5
    table = emb[:seq_len]
    rows = table.shape[0]
    n_chunks = rows // _CHUNK_ROWS
    body = functools.partial(_pipeline_kernel, n_chunks=n_chunks, scale=scale)
    return pl.pallas_call(
        body,
        in_specs=[pl.BlockSpec(memory_space=pl.ANY)],
        out_specs=pl.BlockSpec(memory_space=pl.ANY),
        out_shape=jax.ShapeDtypeStruct((rows, dim), emb.dtype),
        scratch_shapes=(
            [pltpu.VMEM((_CHUNK_ROWS, dim), jnp.float32)] * (2 * _NBUF)
            + [pltpu.SemaphoreType.DMA] * (2 * _NBUF)
        ),
    )(table)
